# baseline (speedup 1.0000x reference)
import functools

import jax
import jax.numpy as jnp
from jax import lax
from jax.experimental import pallas as pl
from jax.experimental.pallas import tpu as pltpu
from jax.experimental.pallas import tpu_sc as plsc

VOCAB = 100000
DIM = 2048
N_HEADS = 16
HEAD_DIM = DIM // N_HEADS
BATCH = 4
SEQ = 4096
TOKENS = BATCH * SEQ
NW = 32
PER_W = TOKENS // NW
CHUNK = 16
NCH = PER_W // CHUNK
NBUF = 3


def _sc_gather(tokens_flat, table):
    mesh = plsc.VectorSubcoreMesh(core_axis_name="c", subcore_axis_name="s")

    @functools.partial(
        pl.kernel,
        mesh=mesh,
        out_type=jax.ShapeDtypeStruct((TOKENS, DIM), jnp.float32),
        scratch_types=[
            pltpu.VMEM((PER_W,), jnp.int32),
            pltpu.VMEM((NBUF, CHUNK, DIM), jnp.float32),
            pltpu.SemaphoreType.DMA,
            pltpu.SemaphoreType.DMA,
            pltpu.SemaphoreType.DMA,
            pltpu.SemaphoreType.DMA,
            pltpu.SemaphoreType.DMA,
            pltpu.SemaphoreType.DMA,
        ],
    )
    def k(idx_hbm, table_hbm, out_hbm, idx_v, rows_v, g0, g1, g2, o0, o1, o2):
        wid = lax.axis_index("s") * 2 + lax.axis_index("c")
        base = wid * PER_W
        pltpu.sync_copy(idx_hbm.at[pl.ds(base, PER_W)], idx_v)
        gsem, osem = (g0, g1, g2), (o0, o1, o2)

        def start_gather(g):
            b = g % NBUF
            return pltpu.async_copy(
                table_hbm.at[idx_v.at[pl.ds(g * CHUNK, CHUNK)]],
                rows_v.at[b], gsem[b])

        def start_out(g):
            b = g % NBUF
            return pltpu.async_copy(
                rows_v.at[b], out_hbm.at[pl.ds(base + g * CHUNK, CHUNK)],
                osem[b])

        gat_cp = [None] * NCH
        out_cp = [None] * NCH
        for g in range(NBUF):
            gat_cp[g] = start_gather(g)
        for g in range(NCH):
            gat_cp[g].wait()
            out_cp[g] = start_out(g)
            p = g - 1
            if p >= 0 and p + NBUF < NCH:
                out_cp[p].wait()
                gat_cp[p + NBUF] = start_gather(p + NBUF)
        for g in range(NCH - NBUF, NCH):
            if g >= 0:
                out_cp[g].wait()

    return k(tokens_flat, table)


TC_ROWS = 2048
TC_K = 8


def _tc_gather(ids, table):
    t3 = table.reshape(VOCAB, 16, DIM // 16)

    def body(ids_ref, *refs):
        ins, out = refs[:TC_K], refs[TC_K]
        for j in range(TC_K):
            out[j, :, :] = ins[j][0, :, :]

    in_specs = [
        pl.BlockSpec(
            (1, 16, DIM // 16),
            (lambda i, ids_ref, j=j: (ids_ref[TC_K * i + j], 0, 0)),
        )
        for j in range(TC_K)
    ]
    out = pl.pallas_call(
        body,
        grid_spec=pltpu.PrefetchScalarGridSpec(
            num_scalar_prefetch=1,
            grid=(TC_ROWS // TC_K,),
            in_specs=in_specs,
            out_specs=pl.BlockSpec((TC_K, 16, DIM // 16), lambda i, ids_ref: (i, 0, 0)),
        ),
        out_shape=jax.ShapeDtypeStruct((TC_ROWS, 16, DIM // 16), jnp.float32),
    )(ids, *([t3] * TC_K))
    return out.reshape(TC_ROWS, DIM)


ASM_B = 512


def _assemble(full, part):
    def body(full_ref, part_ref, out_ref):
        out_ref[...] = part_ref[...]

    return pl.pallas_call(
        body,
        grid=(TC_ROWS // ASM_B,),
        in_specs=[
            pl.BlockSpec(memory_space=pl.ANY),
            pl.BlockSpec((ASM_B, DIM), lambda i: (i, 0)),
        ],
        out_specs=pl.BlockSpec((ASM_B, DIM), lambda i: (i, 0)),
        out_shape=jax.ShapeDtypeStruct((TOKENS, DIM), jnp.float32),
        input_output_aliases={0: 0},
    )(full, part)


ROWB = 512


def _gen_body(cos_ref, sin_ref, mask_ref):
    i = pl.program_id(0)
    t = (lax.broadcasted_iota(jnp.int32, (ROWB, HEAD_DIM), 0) + i * ROWB).astype(
        jnp.float32
    )
    j = lax.broadcasted_iota(jnp.int32, (ROWB, HEAD_DIM), 1)
    half = jnp.where(j < HEAD_DIM // 2, j, j - HEAD_DIM // 2).astype(jnp.float32)
    inv_freq = jnp.exp(half * (-2.0 / HEAD_DIM) * jnp.log(10000.0))
    ang = t * inv_freq
    cos_ref[0] = jnp.cos(ang)
    sin_ref[0] = jnp.sin(ang)
    r = lax.broadcasted_iota(jnp.int32, (ROWB, SEQ), 0) + i * ROWB
    c = lax.broadcasted_iota(jnp.int32, (ROWB, SEQ), 1)
    mask_ref[0, 0] = jnp.where(c > r, -jnp.inf, 0.0).astype(jnp.float32)


def _gen_cos_sin_mask():
    return pl.pallas_call(
        _gen_body,
        grid=(SEQ // ROWB,),
        out_specs=(
            pl.BlockSpec((1, ROWB, HEAD_DIM), lambda i: (0, i, 0)),
            pl.BlockSpec((1, ROWB, HEAD_DIM), lambda i: (0, i, 0)),
            pl.BlockSpec((1, 1, ROWB, SEQ), lambda i: (0, 0, i, 0)),
        ),
        out_shape=(
            jax.ShapeDtypeStruct((1, SEQ, HEAD_DIM), jnp.float32),
            jax.ShapeDtypeStruct((1, SEQ, HEAD_DIM), jnp.float32),
            jax.ShapeDtypeStruct((1, 1, SEQ, SEQ), jnp.float32),
        ),
    )()


def kernel(tokens, W):
    bsz, seq_len = tokens.shape
    flat = tokens.reshape(bsz * seq_len)
    cos, sin, mask = _gen_cos_sin_mask()
    sc_full = _sc_gather(flat, W)
    tc_part = _tc_gather(flat[:TC_ROWS], W)
    hidden = _assemble(sc_full, tc_part).reshape(bsz, seq_len, DIM)
    return (hidden, cos, sin, mask)


# speedup vs baseline: 6.4977x; 6.4977x over previous
import functools

import jax
import jax.numpy as jnp
from jax import lax
from jax.experimental import pallas as pl
from jax.experimental.pallas import tpu as pltpu
from jax.experimental.pallas import tpu_sc as plsc

VOCAB = 100000
DIM = 2048
N_HEADS = 16
HEAD_DIM = DIM // N_HEADS
BATCH = 4
SEQ = 4096
TOKENS = BATCH * SEQ
NW = 32
PER_W = TOKENS // NW
W_PER_B = SEQ // PER_W
CHUNK = 16
NCH = PER_W // CHUNK
NBUF = 3


def _sc_gather(tokens, table):
    mesh = plsc.VectorSubcoreMesh(core_axis_name="c", subcore_axis_name="s")

    @functools.partial(
        pl.kernel,
        mesh=mesh,
        out_type=jax.ShapeDtypeStruct((TOKENS, DIM), jnp.float32),
        scratch_types=[
            pltpu.VMEM((PER_W,), jnp.int32),
            pltpu.VMEM((NBUF, CHUNK, DIM), jnp.float32),
            pltpu.SemaphoreType.DMA,
            pltpu.SemaphoreType.DMA,
            pltpu.SemaphoreType.DMA,
            pltpu.SemaphoreType.DMA,
            pltpu.SemaphoreType.DMA,
            pltpu.SemaphoreType.DMA,
        ],
    )
    def k(idx_hbm, table_hbm, out_hbm, idx_v, rows_v, g0, g1, g2, o0, o1, o2):
        wid = lax.axis_index("s") * 2 + lax.axis_index("c")
        base = wid * PER_W
        pltpu.sync_copy(
            idx_hbm.at[wid // W_PER_B, pl.ds((wid % W_PER_B) * PER_W, PER_W)],
            idx_v)
        gsem, osem = (g0, g1, g2), (o0, o1, o2)

        def start_gather(g):
            b = g % NBUF
            return pltpu.async_copy(
                table_hbm.at[idx_v.at[pl.ds(g * CHUNK, CHUNK)]],
                rows_v.at[b], gsem[b])

        def start_out(g):
            b = g % NBUF
            return pltpu.async_copy(
                rows_v.at[b], out_hbm.at[pl.ds(base + g * CHUNK, CHUNK)],
                osem[b])

        gat_cp = [None] * NCH
        out_cp = [None] * NCH
        for g in range(NBUF):
            gat_cp[g] = start_gather(g)
        for g in range(NCH):
            gat_cp[g].wait()
            out_cp[g] = start_out(g)
            p = g - 1
            if p >= 0 and p + NBUF < NCH:
                out_cp[p].wait()
                gat_cp[p + NBUF] = start_gather(p + NBUF)
        for g in range(NCH - NBUF, NCH):
            if g >= 0:
                out_cp[g].wait()

    return k(tokens, table)


ROWB = 512


def _gen_body(cos_ref, sin_ref, mask_ref):
    i = pl.program_id(0)
    t = (lax.broadcasted_iota(jnp.int32, (ROWB, HEAD_DIM), 0) + i * ROWB).astype(
        jnp.float32
    )
    j = lax.broadcasted_iota(jnp.int32, (ROWB, HEAD_DIM), 1)
    half = jnp.where(j < HEAD_DIM // 2, j, j - HEAD_DIM // 2).astype(jnp.float32)
    inv_freq = jnp.exp(half * (-2.0 / HEAD_DIM) * jnp.log(10000.0))
    ang = t * inv_freq
    cos_ref[0] = jnp.cos(ang)
    sin_ref[0] = jnp.sin(ang)
    r = lax.broadcasted_iota(jnp.int32, (ROWB, SEQ), 0) + i * ROWB
    c = lax.broadcasted_iota(jnp.int32, (ROWB, SEQ), 1)
    mask_ref[0, 0] = jnp.where(c > r, -jnp.inf, 0.0).astype(jnp.float32)


def _gen_cos_sin_mask():
    return pl.pallas_call(
        _gen_body,
        grid=(SEQ // ROWB,),
        out_specs=(
            pl.BlockSpec((1, ROWB, HEAD_DIM), lambda i: (0, i, 0)),
            pl.BlockSpec((1, ROWB, HEAD_DIM), lambda i: (0, i, 0)),
            pl.BlockSpec((1, 1, ROWB, SEQ), lambda i: (0, 0, i, 0)),
        ),
        out_shape=(
            jax.ShapeDtypeStruct((1, SEQ, HEAD_DIM), jnp.float32),
            jax.ShapeDtypeStruct((1, SEQ, HEAD_DIM), jnp.float32),
            jax.ShapeDtypeStruct((1, 1, SEQ, SEQ), jnp.float32),
        ),
    )()


def kernel(tokens, W):
    bsz, seq_len = tokens.shape
    cos, sin, mask = _gen_cos_sin_mask()
    hidden = _sc_gather(tokens, W).reshape(bsz, seq_len, DIM)
    return (hidden, cos, sin, mask)
